# Initial kernel scaffold; baseline (speedup 1.0000x reference)
import jax
import jax.numpy as jnp
from jax.experimental import pallas as pl


def kernel(annotator, posterior_mu, posterior_covtril, eps):
    raise NotImplementedError("write your pallas kernel here")


# speedup vs baseline: 5.7238x; 5.7238x over previous
import functools

import jax
import jax.numpy as jnp
from jax import lax
from jax.experimental import pallas as pl
from jax.experimental.pallas import tpu as pltpu
from jax.experimental.pallas import tpu_sc as plsc

D = 16
CHUNK = 128


def _splat(v):
    return jnp.full((16,), v, jnp.int32)


@functools.cache
def _build_kernel(B, A):
    info = plsc.get_sparse_core_info()
    NC, NS = info.num_cores, info.num_subcores
    NW = NC * NS
    b_per_w = B // NW
    n_chunks = b_per_w // CHUNK
    mesh = plsc.VectorSubcoreMesh(core_axis_name="c", subcore_axis_name="s")

    @functools.partial(
        pl.kernel,
        mesh=mesh,
        compiler_params=pltpu.CompilerParams(
            needs_layout_passes=False, use_tc_tiling_on_sc=False),
        out_type=jax.ShapeDtypeStruct((B * D,), jnp.float32),
        scratch_types=[
            pltpu.VMEM((n_chunks, CHUNK), jnp.int32),
            pltpu.VMEM((CHUNK * D,), jnp.float32),
            pltpu.VMEM((CHUNK, D), jnp.float32),
            pltpu.VMEM((CHUNK, D * D), jnp.float32),
            pltpu.VMEM((CHUNK * D,), jnp.float32),
            pltpu.SemaphoreType.DMA,
            pltpu.SemaphoreType.DMA,
        ],
    )
    def sc_kernel(idx_hbm, mu_hbm, cov_hbm, eps_hbm, out_hbm,
                  idx_v, eps_v, mu_v, cov_v, out_v, sem_a, sem_b):
        wid = lax.axis_index("s") * NC + lax.axis_index("c")
        pltpu.sync_copy(idx_hbm.at[pl.ds(wid * n_chunks, n_chunks)], idx_v)
        iota = lax.iota(jnp.int32, 16)
        iota16 = iota * 16

        for c in range(n_chunks):
            gbase = wid * b_per_w + c * CHUNK
            cov_cp = pltpu.async_copy(cov_hbm.at[idx_v.at[c]], cov_v, sem_a)
            mu_cp = pltpu.async_copy(mu_hbm.at[idx_v.at[c]], mu_v, sem_b)
            pltpu.sync_copy(eps_hbm.at[pl.ds(gbase * D, CHUNK * D)], eps_v)
            mu_cp.wait()
            cov_cp.wait()

            def group(g, carry):
                row = iota + g * 16
                flat = iota16 + g * 256
                acc = [jnp.zeros((16,), jnp.float32) for _ in range(D)]
                for j in range(D):
                    e_j = plsc.load_gather(eps_v, [flat + j])
                    for i in range(j, D):
                        col = plsc.load_gather(cov_v, [row, _splat(i * D + j)])
                        acc[i] = acc[i] + col * e_j
                for i in range(D):
                    plsc.store_scatter(out_v, [flat + i], acc[i])
                return carry

            lax.fori_loop(0, CHUNK // 16, group, 0)

            def add_mu(lb, carry):
                z = out_v[pl.ds(lb * D, D)] + mu_v[lb]
                out_v[pl.ds(lb * D, D)] = z
                return carry

            lax.fori_loop(0, CHUNK, add_mu, 0)
            pltpu.sync_copy(out_v, out_hbm.at[pl.ds(gbase * D, CHUNK * D)])

    return sc_kernel, NW, n_chunks


def kernel(annotator, posterior_mu, posterior_covtril, eps):
    B = annotator.shape[0]
    A = posterior_mu.shape[0]
    sc_kernel, NW, n_chunks = _build_kernel(B, A)
    idx2d = annotator.astype(jnp.int32).reshape(NW * n_chunks, CHUNK)
    cov2d = posterior_covtril.reshape(A, D * D)
    eps_flat = eps.reshape(B * D)
    out = sc_kernel(idx2d, posterior_mu, cov2d, eps_flat)
    return out.reshape(B, D)


# speedup vs baseline: 15.0216x; 2.6244x over previous
import functools

import jax
import jax.numpy as jnp
import numpy as np
from jax import lax
from jax.experimental import pallas as pl
from jax.experimental.pallas import tpu as pltpu
from jax.experimental.pallas import tpu_sc as plsc

D = 16
NCHUNK = 8
MAXU = 5
TBL = 32

ROWS_PER_CORE = {0: [15, 14, 11, 8, 7, 4, 1, 0],
                 1: [13, 12, 10, 9, 6, 5, 3, 2]}


def _assignment():
    table = np.zeros((2, 16, TBL), dtype=np.int32)
    for c in (0, 1):
        rows = ROWS_PER_CORE[c]
        slot_of = {r: s for s, r in enumerate(rows)}
        mu_units = [(1, r, 0, slot_of[r]) for r in rows]
        cov_units = [(0, i * D + j, j, slot_of[i])
                     for i in rows for j in range(i + 1)]
        tiles = [[] for _ in range(16)]
        for t, u in enumerate(mu_units):
            tiles[t].append(u)
        for u in cov_units:
            t = min(range(16), key=lambda t: len(tiles[t]))
            tiles[t].append(u)
        for t in range(16):
            us = tiles[t]
            n_mu = sum(1 for uu in us if uu[0] == 1)
            assert n_mu <= 1 and len(us) <= MAXU, (t, len(us))
            table[c, t, 0] = n_mu
            table[c, t, 1] = len(us)
            for ui, (kind, prow, erow, slot) in enumerate(us):
                table[c, t, 8 + ui * 4 + 0] = kind
                table[c, t, 8 + ui * 4 + 1] = prow
                table[c, t, 8 + ui * 4 + 2] = erow
                table[c, t, 8 + ui * 4 + 3] = slot
    return table


@functools.cache
def _build_kernel(B, A):
    info = plsc.get_sparse_core_info()
    NC, NS = info.num_cores, info.num_subcores
    assert NC == 2 and NS == 16
    chunk = B // NCHUNK
    seg = chunk // 16
    segs_per_slot = B // seg
    table_np = _assignment()
    mesh = plsc.VectorSubcoreMesh(core_axis_name="c", subcore_axis_name="s")

    @functools.partial(
        pl.kernel,
        mesh=mesh,
        compiler_params=pltpu.CompilerParams(needs_layout_passes=False),
        out_type=jax.ShapeDtypeStruct((D * segs_per_slot, seg), jnp.float32),
        scratch_types=[
            pltpu.VMEM((A,), jnp.float32),
            pltpu.VMEM((chunk,), jnp.int32),
            pltpu.VMEM((chunk,), jnp.int32),
            pltpu.VMEM((chunk,), jnp.float32),
            pltpu.VMEM((chunk,), jnp.float32),
            pltpu.VMEM((16, seg), jnp.float32),
            pltpu.VMEM((16, seg), jnp.float32),
            pltpu.VMEM((TBL,), jnp.int32),
            pltpu.VMEM_SHARED((8 * segs_per_slot, seg), jnp.float32),
            pltpu.SemaphoreType.DMA,
            pltpu.SemaphoreType.DMA,
            pltpu.SemaphoreType.DMA,
            pltpu.SemaphoreType.DMA,
            pltpu.SemaphoreType.DMA,
            pltpu.SemaphoreType.DMA,
        ],
    )
    def sc_kernel(tbl_hbm, ann_hbm, mut_hbm, covt_hbm, epst_hbm, out_hbm,
                  plane_v, ann0_v, ann1_v, eps0_v, eps1_v, acc0_v, acc1_v,
                  tbl_v, zacc, sem_p, sem_an0, sem_an1, sem_e0, sem_e1,
                  sem_a):
        cid = lax.axis_index("c")
        sid = lax.axis_index("s")
        pltpu.sync_copy(tbl_hbm.at[pl.ds((cid * 16 + sid) * TBL, TBL)], tbl_v)
        tv0 = tbl_v[pl.ds(0, 16)]
        tv1 = tbl_v[pl.ds(16, 16)]
        n_mu = tv0[0]
        n_units = tv0[1]
        ann_bufs = (ann0_v, ann1_v)
        ann_sems = (sem_an0, sem_an1)
        eps_bufs = (eps0_v, eps1_v)
        eps_sems = (sem_e0, sem_e1)
        acc_bufs = (acc0_v, acc1_v)
        iota = lax.iota(jnp.int32, 16)

        def field(u, f):
            w = 8 + 4 * u + f
            return tv0[w] if w < 16 else tv1[w - 16]

        def run_unit(u, is_mu):
            prow = field(u, 1)
            erow = field(u, 2)
            slot = field(u, 3)
            src = mut_hbm if is_mu else covt_hbm
            plane_cp = pltpu.async_copy(src.at[prow], plane_v, sem_p)
            ann_cps = [None] * NCHUNK
            eps_cps = [None] * NCHUNK
            add_cps = [None] * NCHUNK

            def fetch(c):
                b = c % 2
                ann_cps[c] = pltpu.async_copy(
                    ann_hbm.at[pl.ds(c * chunk, chunk)], ann_bufs[b],
                    ann_sems[b])
                if not is_mu:
                    eps_cps[c] = pltpu.async_copy(
                        epst_hbm.at[erow, pl.ds(c * chunk, chunk)],
                        eps_bufs[b], eps_sems[b])

            fetch(0)
            plane_cp.wait()
            for c in range(NCHUNK):
                if c + 1 < NCHUNK:
                    fetch(c + 1)
                ann_cps[c].wait()
                if not is_mu:
                    eps_cps[c].wait()
                ann = ann_bufs[c % 2]
                eps = eps_bufs[c % 2]
                acc = acc_bufs[c % 2]

                def seg_loop(r, carry2):
                    def group(q, carry3):
                        idx = ann[pl.ds(r * seg + q * 16, 16)]
                        p = plsc.load_gather(plane_v, [idx])
                        if not is_mu:
                            p = p * eps[pl.ds(r * seg + q * 16, 16)]
                        acc[r, pl.ds(q * 16, 16)] = p
                        return carry3

                    lax.fori_loop(0, seg // 16, group, 0)
                    return carry2

                lax.fori_loop(0, 16, seg_loop, 0)
                if c >= 1:
                    add_cps[c - 1].wait()
                base = slot * segs_per_slot + c * 16
                if is_mu:
                    add_cps[c] = pltpu.async_copy(
                        acc, zacc.at[pl.ds(base, 16)], sem_a)
                else:
                    add_cps[c] = pltpu.async_copy(
                        acc, zacc.at[base + iota], sem_a, add=True)
            add_cps[NCHUNK - 1].wait()

        @pl.when(n_mu > 0)
        def _():
            run_unit(0, True)

        plsc.subcore_barrier()
        for u in range(1, MAXU):
            @pl.when(jnp.logical_and(u >= n_mu, u < n_units))
            def _(u=u):
                run_unit(u, False)
        @pl.when(n_mu == 0)
        def _():
            run_unit(0, False)

        plsc.subcore_barrier()

        @pl.when(n_mu > 0)
        def _():
            row = field(0, 1)
            slot = field(0, 3)
            pltpu.sync_copy(
                zacc.at[pl.ds(slot * segs_per_slot, segs_per_slot)],
                out_hbm.at[pl.ds(row * segs_per_slot, segs_per_slot)])

    return sc_kernel, jnp.asarray(table_np.reshape(-1))


def kernel(annotator, posterior_mu, posterior_covtril, eps):
    B = annotator.shape[0]
    A = posterior_mu.shape[0]
    sc_kernel, tbl = _build_kernel(B, A)
    covt = posterior_covtril.transpose(1, 2, 0).reshape(D * D, A)
    mut = posterior_mu.T
    epst = eps.T
    out = sc_kernel(tbl, annotator.astype(jnp.int32), mut, covt, epst)
    return out.reshape(D, B).T


# speedup vs baseline: 17.2563x; 1.1488x over previous
import functools

import jax
import jax.numpy as jnp
import numpy as np
from jax import lax
from jax.experimental import pallas as pl
from jax.experimental.pallas import tpu as pltpu
from jax.experimental.pallas import tpu_sc as plsc

D = 16
NCHUNK = 8
MAXU = 5
TBL = 32

ROWS_PER_CORE = {0: [15, 14, 11, 8, 7, 4, 1, 0],
                 1: [13, 12, 10, 9, 6, 5, 3, 2]}


def _assignment():
    table = np.zeros((2, 16, TBL), dtype=np.int32)
    for c in (0, 1):
        rows = ROWS_PER_CORE[c]
        slot_of = {r: s for s, r in enumerate(rows)}
        mu_units = [(1, r, 0, slot_of[r]) for r in rows]
        cov_units = [(0, i * D + j, j, slot_of[i])
                     for i in rows for j in range(i + 1)]
        tiles = [[] for _ in range(16)]
        for t, u in enumerate(mu_units):
            tiles[t].append(u)
        for u in cov_units:
            t = min(range(16), key=lambda t: len(tiles[t]))
            tiles[t].append(u)
        for t in range(16):
            us = tiles[t]
            n_mu = sum(1 for uu in us if uu[0] == 1)
            assert n_mu <= 1 and len(us) <= MAXU, (t, len(us))
            table[c, t, 0] = n_mu
            table[c, t, 1] = len(us)
            for ui, (kind, prow, erow, slot) in enumerate(us):
                table[c, t, 8 + ui * 4 + 0] = kind
                table[c, t, 8 + ui * 4 + 1] = prow
                table[c, t, 8 + ui * 4 + 2] = erow
                table[c, t, 8 + ui * 4 + 3] = slot
    return table


@functools.cache
def _build_kernel(B, A):
    info = plsc.get_sparse_core_info()
    NC, NS = info.num_cores, info.num_subcores
    assert NC == 2 and NS == 16
    chunk = B // NCHUNK
    seg = chunk // 16
    segs_per_slot = B // seg
    table_np = _assignment()
    mesh = plsc.VectorSubcoreMesh(core_axis_name="c", subcore_axis_name="s")

    @functools.partial(
        pl.kernel,
        mesh=mesh,
        compiler_params=pltpu.CompilerParams(needs_layout_passes=False),
        out_type=jax.ShapeDtypeStruct((D * segs_per_slot, seg), jnp.float32),
        scratch_types=[
            pltpu.VMEM((A,), jnp.float32),
            pltpu.VMEM((chunk,), jnp.int32),
            pltpu.VMEM((chunk,), jnp.int32),
            pltpu.VMEM((chunk,), jnp.float32),
            pltpu.VMEM((chunk,), jnp.float32),
            pltpu.VMEM((16, seg), jnp.float32),
            pltpu.VMEM((16, seg), jnp.float32),
            pltpu.VMEM((TBL,), jnp.int32),
            pltpu.VMEM_SHARED((8 * segs_per_slot, seg), jnp.float32),
            pltpu.SemaphoreType.DMA,
            pltpu.SemaphoreType.DMA,
            pltpu.SemaphoreType.DMA,
            pltpu.SemaphoreType.DMA,
            pltpu.SemaphoreType.DMA,
            pltpu.SemaphoreType.DMA,
        ],
    )
    def sc_kernel(tbl_hbm, ann_hbm, mut_hbm, covt_hbm, epst_hbm, out_hbm,
                  plane_v, ann0_v, ann1_v, eps0_v, eps1_v, acc0_v, acc1_v,
                  tbl_v, zacc, sem_p, sem_an0, sem_an1, sem_e0, sem_e1,
                  sem_a):
        cid = lax.axis_index("c")
        sid = lax.axis_index("s")
        pltpu.sync_copy(tbl_hbm.at[pl.ds((cid * 16 + sid) * TBL, TBL)], tbl_v)
        tv0 = tbl_v[pl.ds(0, 16)]
        tv1 = tbl_v[pl.ds(16, 16)]
        n_mu = tv0[0]
        n_units = tv0[1]
        ann_bufs = (ann0_v, ann1_v)
        ann_sems = (sem_an0, sem_an1)
        eps_bufs = (eps0_v, eps1_v)
        eps_sems = (sem_e0, sem_e1)
        acc_bufs = (acc0_v, acc1_v)
        iota = lax.iota(jnp.int32, 16)

        def field(u, f):
            w = 8 + 4 * u + f
            return tv0[w] if w < 16 else tv1[w - 16]

        def run_unit(u, is_mu):
            prow = field(u, 1)
            erow = field(u, 2)
            slot = field(u, 3)
            src = mut_hbm if is_mu else covt_hbm
            plane_cps = [pltpu.async_copy(src.at[prow], plane_v, sem_p)]
            ann_cps = [None] * NCHUNK
            eps_cps = [None] * NCHUNK
            add_cps = [None] * NCHUNK

            def fetch(c):
                b = c % 2
                ann_cps[c] = pltpu.async_copy(
                    ann_hbm.at[pl.ds(c * chunk, chunk)], ann_bufs[b],
                    ann_sems[b])
                if not is_mu:
                    eps_cps[c] = pltpu.async_copy(
                        epst_hbm.at[erow, pl.ds(c * chunk, chunk)],
                        eps_bufs[b], eps_sems[b])

            fetch(0)
            for cp in plane_cps:
                cp.wait()
            for c in range(NCHUNK):
                if c + 1 < NCHUNK:
                    fetch(c + 1)
                ann_cps[c].wait()
                if not is_mu:
                    eps_cps[c].wait()
                ann = ann_bufs[c % 2]
                eps = eps_bufs[c % 2]
                acc = acc_bufs[c % 2]

                def seg_loop(r, carry2):
                    nq = seg // 16
                    idxs = [ann[pl.ds(r * seg + q * 16, 16)]
                            for q in range(nq)]
                    ps = [plsc.load_gather(plane_v, [idxs[q]])
                          for q in range(nq)]
                    for q in range(nq):
                        p = ps[q]
                        if not is_mu:
                            p = p * eps[pl.ds(r * seg + q * 16, 16)]
                        acc[r, pl.ds(q * 16, 16)] = p
                    return carry2

                lax.fori_loop(0, 16, seg_loop, 0)
                if c >= 1:
                    add_cps[c - 1].wait()
                base = slot * segs_per_slot + c * 16
                if is_mu:
                    add_cps[c] = pltpu.async_copy(
                        acc, zacc.at[pl.ds(base, 16)], sem_a)
                else:
                    add_cps[c] = pltpu.async_copy(
                        acc, zacc.at[base + iota], sem_a, add=True)
            add_cps[NCHUNK - 1].wait()

        @pl.when(n_mu > 0)
        def _():
            run_unit(0, True)

        plsc.subcore_barrier()
        for u in range(1, MAXU):
            @pl.when(jnp.logical_and(u >= n_mu, u < n_units))
            def _(u=u):
                run_unit(u, False)
        @pl.when(n_mu == 0)
        def _():
            run_unit(0, False)

        plsc.subcore_barrier()

        @pl.when(n_mu > 0)
        def _():
            row = field(0, 1)
            slot = field(0, 3)
            pltpu.sync_copy(
                zacc.at[pl.ds(slot * segs_per_slot, segs_per_slot)],
                out_hbm.at[pl.ds(row * segs_per_slot, segs_per_slot)])

    return sc_kernel, jnp.asarray(table_np.reshape(-1))


def kernel(annotator, posterior_mu, posterior_covtril, eps):
    B = annotator.shape[0]
    A = posterior_mu.shape[0]
    sc_kernel, tbl = _build_kernel(B, A)
    covt = posterior_covtril.transpose(1, 2, 0).reshape(D * D, A)
    mut = posterior_mu.T
    epst = eps.T
    out = sc_kernel(tbl, annotator.astype(jnp.int32), mut, covt, epst)
    return out.reshape(D, B).T


# speedup vs baseline: 18.1204x; 1.0501x over previous
import functools

import jax
import jax.numpy as jnp
import numpy as np
from jax import lax
from jax.experimental import pallas as pl
from jax.experimental.pallas import tpu as pltpu
from jax.experimental.pallas import tpu_sc as plsc

D = 16
NCHUNK = 8
MAXU = 5
TBL = 32

ROWS_PER_CORE = {0: [15, 14, 11, 8, 7, 4, 1, 0],
                 1: [13, 12, 10, 9, 6, 5, 3, 2]}


def _assignment():
    table = np.zeros((2, 16, TBL), dtype=np.int32)
    for c in (0, 1):
        rows = ROWS_PER_CORE[c]
        slot_of = {r: s for s, r in enumerate(rows)}
        units = [(1, r, 0, slot_of[r]) for r in rows]
        units += [(0, i * D + j, j, slot_of[i])
                  for i in rows for j in range(i + 1)]
        tiles = [[] for _ in range(16)]
        for u in units:
            t = min(range(16), key=lambda t: len(tiles[t]))
            tiles[t].append(u)
        for t in range(16):
            us = tiles[t]
            assert len(us) <= MAXU, (t, len(us))
            table[c, t, 0] = len(us)
            table[c, t, 1] = rows[t] if t < 8 else 0
            for ui, (kind, prow, erow, slot) in enumerate(us):
                table[c, t, 8 + ui * 4 + 0] = kind
                table[c, t, 8 + ui * 4 + 1] = prow
                table[c, t, 8 + ui * 4 + 2] = erow
                table[c, t, 8 + ui * 4 + 3] = slot
    return table


@functools.cache
def _build_kernel(B, A):
    info = plsc.get_sparse_core_info()
    NC, NS = info.num_cores, info.num_subcores
    assert NC == 2 and NS == 16
    chunk = B // NCHUNK
    seg = chunk // 16
    segs_per_slot = B // seg
    rows_per_tile = 8 * segs_per_slot // 16
    table_np = _assignment()
    mesh = plsc.VectorSubcoreMesh(core_axis_name="c", subcore_axis_name="s")

    @functools.partial(
        pl.kernel,
        mesh=mesh,
        compiler_params=pltpu.CompilerParams(needs_layout_passes=False),
        out_type=jax.ShapeDtypeStruct((D * segs_per_slot, seg), jnp.float32),
        scratch_types=[
            pltpu.VMEM((A,), jnp.float32),
            pltpu.VMEM((chunk,), jnp.int32),
            pltpu.VMEM((chunk,), jnp.int32),
            pltpu.VMEM((chunk,), jnp.float32),
            pltpu.VMEM((chunk,), jnp.float32),
            pltpu.VMEM((16, seg), jnp.float32),
            pltpu.VMEM((16, seg), jnp.float32),
            pltpu.VMEM((TBL,), jnp.int32),
            pltpu.VMEM_SHARED((8 * segs_per_slot, seg), jnp.float32),
            pltpu.SemaphoreType.DMA,
            pltpu.SemaphoreType.DMA,
            pltpu.SemaphoreType.DMA,
            pltpu.SemaphoreType.DMA,
            pltpu.SemaphoreType.DMA,
            pltpu.SemaphoreType.DMA,
        ],
    )
    def sc_kernel(tbl_hbm, ann_hbm, mut_hbm, covt_hbm, epst_hbm, out_hbm,
                  plane_v, ann0_v, ann1_v, eps0_v, eps1_v, acc0_v, acc1_v,
                  tbl_v, zacc, sem_p, sem_an0, sem_an1, sem_e0, sem_e1,
                  sem_a):
        cid = lax.axis_index("c")
        sid = lax.axis_index("s")
        pltpu.sync_copy(tbl_hbm.at[pl.ds((cid * 16 + sid) * TBL, TBL)], tbl_v)
        tv0 = tbl_v[pl.ds(0, 16)]
        tv1 = tbl_v[pl.ds(16, 16)]
        n_units = tv0[0]
        ann_bufs = (ann0_v, ann1_v)
        ann_sems = (sem_an0, sem_an1)
        eps_bufs = (eps0_v, eps1_v)
        eps_sems = (sem_e0, sem_e1)
        acc_bufs = (acc0_v, acc1_v)
        iota = lax.iota(jnp.int32, 16)

        zero = jnp.zeros((16,), jnp.float32)

        def zrow(r, carry):
            for q in range(seg // 16):
                acc0_v[r, pl.ds(q * 16, 16)] = zero
            return carry

        lax.fori_loop(0, 16, zrow, 0)
        zbase = sid * rows_per_tile
        for k in range(rows_per_tile // 16):
            pltpu.sync_copy(acc0_v, zacc.at[pl.ds(zbase + k * 16, 16)])
        plsc.subcore_barrier()

        def field(u, f):
            w = 8 + 4 * u + f
            return tv0[w] if w < 16 else tv1[w - 16]

        def run_unit(u, is_mu):
            prow = field(u, 1)
            erow = field(u, 2)
            slot = field(u, 3)
            src = mut_hbm if is_mu else covt_hbm
            plane_cp = pltpu.async_copy(src.at[prow], plane_v, sem_p)
            ann_cps = [None] * NCHUNK
            eps_cps = [None] * NCHUNK
            add_cps = [None] * NCHUNK

            def fetch(c):
                b = c % 2
                ann_cps[c] = pltpu.async_copy(
                    ann_hbm.at[pl.ds(c * chunk, chunk)], ann_bufs[b],
                    ann_sems[b])
                if not is_mu:
                    eps_cps[c] = pltpu.async_copy(
                        epst_hbm.at[erow, pl.ds(c * chunk, chunk)],
                        eps_bufs[b], eps_sems[b])

            fetch(0)
            plane_cp.wait()
            for c in range(NCHUNK):
                if c + 1 < NCHUNK:
                    fetch(c + 1)
                ann_cps[c].wait()
                if not is_mu:
                    eps_cps[c].wait()
                ann = ann_bufs[c % 2]
                eps = eps_bufs[c % 2]
                acc = acc_bufs[c % 2]

                def seg_loop(r, carry2):
                    nq = seg // 16
                    idxs = [ann[pl.ds(r * seg + q * 16, 16)]
                            for q in range(nq)]
                    ps = [plsc.load_gather(plane_v, [idxs[q]])
                          for q in range(nq)]
                    for q in range(nq):
                        p = ps[q]
                        if not is_mu:
                            p = p * eps[pl.ds(r * seg + q * 16, 16)]
                        acc[r, pl.ds(q * 16, 16)] = p
                    return carry2

                lax.fori_loop(0, 16, seg_loop, 0)
                if c >= 1:
                    add_cps[c - 1].wait()
                base = slot * segs_per_slot + c * 16
                add_cps[c] = pltpu.async_copy(
                    acc, zacc.at[base + iota], sem_a, add=True)
            add_cps[NCHUNK - 1].wait()

        for u in range(MAXU):
            kind = field(u, 0)
            live = u < n_units

            @pl.when(jnp.logical_and(live, kind == 1))
            def _(u=u):
                run_unit(u, True)

            @pl.when(jnp.logical_and(live, kind == 0))
            def _(u=u):
                run_unit(u, False)

        plsc.subcore_barrier()

        @pl.when(sid < 8)
        def _():
            row = tv0[1]
            pltpu.sync_copy(
                zacc.at[pl.ds(sid * segs_per_slot, segs_per_slot)],
                out_hbm.at[pl.ds(row * segs_per_slot, segs_per_slot)])

    return sc_kernel, jnp.asarray(table_np.reshape(-1))


def kernel(annotator, posterior_mu, posterior_covtril, eps):
    B = annotator.shape[0]
    A = posterior_mu.shape[0]
    sc_kernel, tbl = _build_kernel(B, A)
    covt = posterior_covtril.transpose(1, 2, 0).reshape(D * D, A)
    mut = posterior_mu.T
    epst = eps.T
    out = sc_kernel(tbl, annotator.astype(jnp.int32), mut, covt, epst)
    return out.reshape(D, B).T


# speedup vs baseline: 18.5716x; 1.0249x over previous
import functools

import jax
import jax.numpy as jnp
import numpy as np
from jax import lax
from jax.experimental import pallas as pl
from jax.experimental.pallas import tpu as pltpu
from jax.experimental.pallas import tpu_sc as plsc

D = 16
NCHUNK = 8
MAXU = 5
TBL = 32

ROWS_PER_CORE = {0: [15, 14, 11, 8, 7, 4, 1, 0],
                 1: [13, 12, 10, 9, 6, 5, 3, 2]}


def _assignment():
    table = np.zeros((2, 16, TBL), dtype=np.int32)
    for c in (0, 1):
        rows = ROWS_PER_CORE[c]
        slot_of = {r: s for s, r in enumerate(rows)}
        units = [(1, r, D, slot_of[r]) for r in rows]
        units += [(0, i * D + j, j, slot_of[i])
                  for i in rows for j in range(i + 1)]
        tiles = [[] for _ in range(16)]
        for u in units:
            t = min(range(16), key=lambda t: len(tiles[t]))
            tiles[t].append(u)
        for t in range(16):
            us = tiles[t]
            assert len(us) <= MAXU, (t, len(us))
            table[c, t, 0] = len(us)
            table[c, t, 1] = rows[t] if t < 8 else 0
            for ui, (kind, prow, erow, slot) in enumerate(us):
                table[c, t, 8 + ui * 4 + 0] = kind
                table[c, t, 8 + ui * 4 + 1] = prow
                table[c, t, 8 + ui * 4 + 2] = erow
                table[c, t, 8 + ui * 4 + 3] = slot
    return table


@functools.cache
def _build_kernel(B, A):
    info = plsc.get_sparse_core_info()
    NC, NS = info.num_cores, info.num_subcores
    assert NC == 2 and NS == 16
    chunk = B // NCHUNK
    seg = chunk // 16
    segs_per_slot = B // seg
    rows_per_tile = 8 * segs_per_slot // 16
    table_np = _assignment()
    mesh = plsc.VectorSubcoreMesh(core_axis_name="c", subcore_axis_name="s")

    @functools.partial(
        pl.kernel,
        mesh=mesh,
        compiler_params=pltpu.CompilerParams(needs_layout_passes=False),
        out_type=jax.ShapeDtypeStruct((D * segs_per_slot, seg), jnp.float32),
        scratch_types=[
            pltpu.VMEM((A,), jnp.float32),
            pltpu.VMEM((chunk,), jnp.int32),
            pltpu.VMEM((chunk,), jnp.int32),
            pltpu.VMEM((chunk,), jnp.float32),
            pltpu.VMEM((chunk,), jnp.float32),
            pltpu.VMEM((16, seg), jnp.float32),
            pltpu.VMEM((16, seg), jnp.float32),
            pltpu.VMEM((TBL,), jnp.int32),
            pltpu.VMEM_SHARED((8 * segs_per_slot, seg), jnp.float32),
            pltpu.SemaphoreType.DMA,
            pltpu.SemaphoreType.DMA,
            pltpu.SemaphoreType.DMA,
            pltpu.SemaphoreType.DMA,
            pltpu.SemaphoreType.DMA,
            pltpu.SemaphoreType.DMA,
        ],
    )
    def sc_kernel(tbl_hbm, ann_hbm, mut_hbm, covt_hbm, epst_hbm, out_hbm,
                  plane_v, ann0_v, ann1_v, eps0_v, eps1_v, acc0_v, acc1_v,
                  tbl_v, zacc, sem_p, sem_an0, sem_an1, sem_e0, sem_e1,
                  sem_a):
        cid = lax.axis_index("c")
        sid = lax.axis_index("s")
        pltpu.sync_copy(tbl_hbm.at[pl.ds((cid * 16 + sid) * TBL, TBL)], tbl_v)
        tv0 = tbl_v[pl.ds(0, 16)]
        tv1 = tbl_v[pl.ds(16, 16)]
        n_units = tv0[0]
        ann_bufs = (ann0_v, ann1_v)
        ann_sems = (sem_an0, sem_an1)
        eps_bufs = (eps0_v, eps1_v)
        eps_sems = (sem_e0, sem_e1)
        acc_bufs = (acc0_v, acc1_v)
        iota = lax.iota(jnp.int32, 16)

        zero = jnp.zeros((16,), jnp.float32)

        def zrow(r, carry):
            for q in range(seg // 16):
                acc0_v[r, pl.ds(q * 16, 16)] = zero
            return carry

        lax.fori_loop(0, 16, zrow, 0)
        zbase = sid * rows_per_tile
        for k in range(rows_per_tile // 16):
            pltpu.sync_copy(acc0_v, zacc.at[pl.ds(zbase + k * 16, 16)])
        plsc.subcore_barrier()

        def field(u, f):
            w = 8 + 4 * u + f
            return tv0[w] if w < 16 else tv1[w - 16]

        def run_unit(u):
            kind = field(u, 0)
            prow = field(u, 1)
            erow = field(u, 2)
            slot = field(u, 3)
            ann_cps = [None] * NCHUNK
            eps_cps = [None] * NCHUNK
            add_cps = [None] * NCHUNK

            def fetch(c):
                b = c % 2
                ann_cps[c] = pltpu.async_copy(
                    ann_hbm.at[pl.ds(c * chunk, chunk)], ann_bufs[b],
                    ann_sems[b])
                eps_cps[c] = pltpu.async_copy(
                    epst_hbm.at[erow, pl.ds(c * chunk, chunk)],
                    eps_bufs[b], eps_sems[b])

            fetch(0)

            @pl.when(kind == 1)
            def _():
                pltpu.sync_copy(mut_hbm.at[prow], plane_v)

            @pl.when(kind == 0)
            def _():
                pltpu.sync_copy(covt_hbm.at[prow], plane_v)

            for c in range(NCHUNK):
                if c + 1 < NCHUNK:
                    fetch(c + 1)
                ann_cps[c].wait()
                eps_cps[c].wait()
                ann = ann_bufs[c % 2]
                eps = eps_bufs[c % 2]
                acc = acc_bufs[c % 2]

                def seg_loop(r, carry2):
                    nq = seg // 16
                    idxs = [ann[pl.ds(r * seg + q * 16, 16)]
                            for q in range(nq)]
                    ps = [plsc.load_gather(plane_v, [idxs[q]])
                          for q in range(nq)]
                    for q in range(nq):
                        acc[r, pl.ds(q * 16, 16)] = (
                            ps[q] * eps[pl.ds(r * seg + q * 16, 16)])
                    return carry2

                lax.fori_loop(0, 16, seg_loop, 0)
                if c >= 1:
                    add_cps[c - 1].wait()
                base = slot * segs_per_slot + c * 16
                add_cps[c] = pltpu.async_copy(
                    acc, zacc.at[base + iota], sem_a, add=True)
            add_cps[NCHUNK - 1].wait()

        for u in range(MAXU):
            @pl.when(u < n_units)
            def _(u=u):
                run_unit(u)

        plsc.subcore_barrier()

        @pl.when(sid < 8)
        def _():
            row = tv0[1]
            pltpu.sync_copy(
                zacc.at[pl.ds(sid * segs_per_slot, segs_per_slot)],
                out_hbm.at[pl.ds(row * segs_per_slot, segs_per_slot)])

    return sc_kernel, jnp.asarray(table_np.reshape(-1))


def kernel(annotator, posterior_mu, posterior_covtril, eps):
    B = annotator.shape[0]
    A = posterior_mu.shape[0]
    sc_kernel, tbl = _build_kernel(B, A)
    covt = posterior_covtril.transpose(1, 2, 0).reshape(D * D, A)
    mut = posterior_mu.T
    epst = jnp.concatenate([eps.T, jnp.ones((1, B), jnp.float32)], axis=0)
    out = sc_kernel(tbl, annotator.astype(jnp.int32), mut, covt, epst)
    return out.reshape(D, B).T
